# Initial kernel scaffold; baseline (speedup 1.0000x reference)
import jax
import jax.numpy as jnp
from jax.experimental import pallas as pl


def kernel(padded_pos, padded_deprel, pos_table, deprel_table):
    raise NotImplementedError("write your pallas kernel here")


# speedup vs baseline: 68.9802x; 68.9802x over previous
import functools

import jax
import jax.numpy as jnp
from jax import lax
from jax.experimental import pallas as pl
from jax.experimental.pallas import tpu as pltpu
from jax.experimental.pallas import tpu_sc as plsc

_NW = 32
_LANES = 16
_C = 128


def _sc_counts(pos_flat, dep_flat, B, L):
    R = B // _NW
    nvec = (L + _LANES - 1) // _LANES
    rem = L - (nvec - 1) * _LANES
    mesh = plsc.VectorSubcoreMesh(core_axis_name="c", subcore_axis_name="s")

    @functools.partial(
        pl.kernel,
        out_type=jax.ShapeDtypeStruct((B * _C,), jnp.float32),
        mesh=mesh,
        compiler_params=pltpu.CompilerParams(needs_layout_passes=False),
        scratch_types=[
            pltpu.VMEM((R * L + _LANES,), jnp.int32),
            pltpu.VMEM((R * L + _LANES,), jnp.int32),
            pltpu.VMEM((R * _C,), jnp.float32),
        ],
    )
    def k(pos_hbm, dep_hbm, out_hbm, pos_v, dep_v, cnt_v):
        wid = lax.axis_index("s") * 2 + lax.axis_index("c")
        base = pl.multiple_of(wid * (R * L), 8)
        pltpu.sync_copy(pos_hbm.at[pl.ds(base, R * L)], pos_v.at[pl.ds(0, R * L)])
        pltpu.sync_copy(dep_hbm.at[pl.ds(base, R * L)], dep_v.at[pl.ds(0, R * L)])

        ones = jnp.full((_LANES,), 1.0, jnp.float32)
        zeros = jnp.zeros((_LANES,), jnp.float32)
        tail = jnp.where(lax.iota(jnp.int32, _LANES) < rem, 1.0, 0.0)

        def row(r, carry):
            rb = r * L
            ro = r * _C
            for j in range(_C // _LANES):
                cnt_v[pl.ds(ro + j * _LANES, _LANES)] = zeros
            for j in range(nvec):
                idx = pos_v[pl.ds(rb + j * _LANES, _LANES)]
                col = jnp.bitwise_and(idx, 63)
                val = ones if j < nvec - 1 else tail
                plsc.addupdate_scatter(cnt_v, [col + ro], val)
            for j in range(nvec):
                idx = dep_v[pl.ds(rb + j * _LANES, _LANES)]
                col = jnp.bitwise_and(idx, 63) + 64
                val = ones if j < nvec - 1 else tail
                plsc.addupdate_scatter(cnt_v, [col + ro], val)
            return carry

        lax.fori_loop(0, R, row, 0)

        obase = pl.multiple_of(wid * (R * _C), 8)
        pltpu.sync_copy(cnt_v, out_hbm.at[pl.ds(obase, R * _C)])

    return k(pos_flat, dep_flat)


def _tc_matmul(counts, w_pos, w_dep, B):
    BLK = 2048

    def body(c_ref, wp_ref, wd_ref, po_ref, do_ref):
        c = c_ref[...]
        po_ref[...] = jnp.dot(c, wp_ref[...], preferred_element_type=jnp.float32)
        do_ref[...] = jnp.dot(c, wd_ref[...], preferred_element_type=jnp.float32)

    return pl.pallas_call(
        body,
        grid=(B // BLK,),
        in_specs=[
            pl.BlockSpec((BLK, _C), lambda i: (i, 0)),
            pl.BlockSpec((_C, 64), lambda i: (0, 0)),
            pl.BlockSpec((_C, 64), lambda i: (0, 0)),
        ],
        out_specs=[
            pl.BlockSpec((BLK, 64), lambda i: (i, 0)),
            pl.BlockSpec((BLK, 64), lambda i: (i, 0)),
        ],
        out_shape=[
            jax.ShapeDtypeStruct((B, 64), jnp.float32),
            jax.ShapeDtypeStruct((B, 64), jnp.float32),
        ],
    )(counts, w_pos, w_dep)


def kernel(padded_pos, padded_deprel, pos_table, deprel_table):
    B, L = padded_pos.shape
    counts = _sc_counts(padded_pos.reshape(-1), padded_deprel.reshape(-1), B, L)
    counts = counts.reshape(B, _C)
    w_pos = jnp.zeros((_C, 64), jnp.float32).at[: pos_table.shape[0]].set(pos_table)
    w_dep = (
        jnp.zeros((_C, 64), jnp.float32)
        .at[64 : 64 + deprel_table.shape[0]]
        .set(deprel_table)
    )
    return tuple(_tc_matmul(counts, w_pos, w_dep, B))


# speedup vs baseline: 74.7012x; 1.0829x over previous
import functools

import jax
import jax.numpy as jnp
from jax import lax
from jax.experimental import pallas as pl
from jax.experimental.pallas import tpu as pltpu
from jax.experimental.pallas import tpu_sc as plsc

_NW = 32
_LANES = 16
_C = 128


def _sc_counts(packed, B):
    R = B // _NW
    CH = 256
    mesh = plsc.VectorSubcoreMesh(core_axis_name="c", subcore_axis_name="s")

    @functools.partial(
        pl.kernel,
        out_type=jax.ShapeDtypeStruct((B, _C), jnp.float32),
        mesh=mesh,
        compiler_params=pltpu.CompilerParams(needs_layout_passes=False),
        scratch_types=[
            pltpu.VMEM((CH, _C), jnp.int32),
            pltpu.VMEM((CH, _C), jnp.float32),
        ],
    )
    def k(pk_hbm, out_hbm, idx_v, cnt_v):
        wid = lax.axis_index("s") * 2 + lax.axis_index("c")
        ones = jnp.full((_LANES,), 1.0, jnp.float32)
        zeros = jnp.zeros((_LANES,), jnp.float32)

        def chunk(ci, carry):
            base = wid * R + ci * CH
            pltpu.sync_copy(pk_hbm.at[pl.ds(base, CH)], idx_v)

            @plsc.parallel_loop(0, CH, unroll=2)
            def row(r):
                rows = jnp.full((_LANES,), 0, jnp.int32) + r
                for j in range(_C // _LANES):
                    cnt_v[r, pl.ds(j * _LANES, _LANES)] = zeros
                for j in range(_C // _LANES):
                    idx = idx_v[r, pl.ds(j * _LANES, _LANES)]
                    plsc.addupdate_scatter(cnt_v, [rows, idx], ones)

            pltpu.sync_copy(cnt_v, out_hbm.at[pl.ds(base, CH)])
            return carry

        lax.fori_loop(0, R // CH, chunk, 0)

    return k(packed)


def _tc_matmul(counts, w_pos, w_dep, B):
    BLK = 2048

    def body(c_ref, wp_ref, wd_ref, po_ref, do_ref):
        c = c_ref[...]
        po_ref[...] = jnp.dot(c, wp_ref[...], preferred_element_type=jnp.float32)
        do_ref[...] = jnp.dot(c, wd_ref[...], preferred_element_type=jnp.float32)

    return pl.pallas_call(
        body,
        grid=(B // BLK,),
        in_specs=[
            pl.BlockSpec((BLK, _C), lambda i: (i, 0)),
            pl.BlockSpec((_C, 64), lambda i: (0, 0)),
            pl.BlockSpec((_C, 64), lambda i: (0, 0)),
        ],
        out_specs=[
            pl.BlockSpec((BLK, 64), lambda i: (i, 0)),
            pl.BlockSpec((BLK, 64), lambda i: (i, 0)),
        ],
        out_shape=[
            jax.ShapeDtypeStruct((B, 64), jnp.float32),
            jax.ShapeDtypeStruct((B, 64), jnp.float32),
        ],
    )(counts, w_pos, w_dep)


def kernel(padded_pos, padded_deprel, pos_table, deprel_table):
    B, L = padded_pos.shape
    fill = jnp.zeros((B, (_C // 2) - L), jnp.int32)
    packed = jnp.concatenate(
        [padded_pos, fill, padded_deprel + 64, fill], axis=1
    )
    counts = _sc_counts(packed, B)
    w_pos = jnp.zeros((_C, 64), jnp.float32).at[: pos_table.shape[0]].set(pos_table)
    w_dep = (
        jnp.zeros((_C, 64), jnp.float32)
        .at[64 : 64 + deprel_table.shape[0]]
        .set(deprel_table)
    )
    return tuple(_tc_matmul(counts, w_pos, w_dep, B))


# speedup vs baseline: 109.3196x; 1.4634x over previous
import functools

import jax
import jax.numpy as jnp
from jax import lax
from jax.experimental import pallas as pl
from jax.experimental.pallas import tpu as pltpu
from jax.experimental.pallas import tpu_sc as plsc

_NW = 32
_LANES = 16
_C = 128


def _sc_counts(padded_pos, padded_deprel, B, L):
    R = B // _NW
    CH = 256
    offs = [j * _LANES for j in range(L // _LANES)] + [L - _LANES]
    novl = L - (L // _LANES) * _LANES
    mesh = plsc.VectorSubcoreMesh(core_axis_name="c", subcore_axis_name="s")

    @functools.partial(
        pl.kernel,
        out_type=jax.ShapeDtypeStruct((B, _C), jnp.float32),
        mesh=mesh,
        compiler_params=pltpu.CompilerParams(needs_layout_passes=False),
        scratch_types=[
            pltpu.VMEM((CH, L), jnp.int32),
            pltpu.VMEM((CH, L), jnp.int32),
            pltpu.VMEM((CH, _C), jnp.float32),
        ],
    )
    def k(pos_hbm, dep_hbm, out_hbm, pos_v, dep_v, cnt_v):
        wid = lax.axis_index("s") * 2 + lax.axis_index("c")
        ones = jnp.full((_LANES,), 1.0, jnp.float32)
        zeros = jnp.zeros((_LANES,), jnp.float32)
        tail = jnp.where(lax.iota(jnp.int32, _LANES) >= _LANES - novl, 1.0, 0.0)
        vals = [ones] * (len(offs) - 1) + [tail]

        def chunk(ci, carry):
            base = wid * R + ci * CH
            pltpu.sync_copy(pos_hbm.at[pl.ds(base, CH)], pos_v)
            pltpu.sync_copy(dep_hbm.at[pl.ds(base, CH)], dep_v)

            @plsc.parallel_loop(0, CH, unroll=2)
            def row(r):
                rows = jnp.full((_LANES,), 0, jnp.int32) + r
                for j in range(_C // _LANES):
                    cnt_v[r, pl.ds(j * _LANES, _LANES)] = zeros
                for off, val in zip(offs, vals):
                    idx = pos_v[r, pl.ds(off, _LANES)]
                    plsc.addupdate_scatter(cnt_v, [rows, idx], val)
                for off, val in zip(offs, vals):
                    idx = dep_v[r, pl.ds(off, _LANES)] + 64
                    plsc.addupdate_scatter(cnt_v, [rows, idx], val)

            pltpu.sync_copy(cnt_v, out_hbm.at[pl.ds(base, CH)])
            return carry

        lax.fori_loop(0, R // CH, chunk, 0)

    return k(padded_pos, padded_deprel)


def _tc_matmul(counts, w_pos, w_dep, B):
    BLK = 2048

    def body(c_ref, wp_ref, wd_ref, po_ref, do_ref):
        c = c_ref[...]
        po_ref[...] = jnp.dot(c, wp_ref[...], preferred_element_type=jnp.float32)
        do_ref[...] = jnp.dot(c, wd_ref[...], preferred_element_type=jnp.float32)

    return pl.pallas_call(
        body,
        grid=(B // BLK,),
        in_specs=[
            pl.BlockSpec((BLK, _C), lambda i: (i, 0)),
            pl.BlockSpec((_C, 64), lambda i: (0, 0)),
            pl.BlockSpec((_C, 64), lambda i: (0, 0)),
        ],
        out_specs=[
            pl.BlockSpec((BLK, 64), lambda i: (i, 0)),
            pl.BlockSpec((BLK, 64), lambda i: (i, 0)),
        ],
        out_shape=[
            jax.ShapeDtypeStruct((B, 64), jnp.float32),
            jax.ShapeDtypeStruct((B, 64), jnp.float32),
        ],
    )(counts, w_pos, w_dep)


def kernel(padded_pos, padded_deprel, pos_table, deprel_table):
    B, L = padded_pos.shape
    counts = _sc_counts(padded_pos, padded_deprel, B, L)
    w_pos = jnp.zeros((_C, 64), jnp.float32).at[: pos_table.shape[0]].set(pos_table)
    w_dep = (
        jnp.zeros((_C, 64), jnp.float32)
        .at[64 : 64 + deprel_table.shape[0]]
        .set(deprel_table)
    )
    return tuple(_tc_matmul(counts, w_pos, w_dep, B))


# speedup vs baseline: 138.1654x; 1.2639x over previous
import functools

import jax
import jax.numpy as jnp
from jax import lax
from jax.experimental import pallas as pl
from jax.experimental.pallas import tpu as pltpu
from jax.experimental.pallas import tpu_sc as plsc

_NW = 32
_LANES = 16
_C = 128


def _sc_counts(pos_t, dep_t, B, L):
    R = B // _NW
    G = R // _LANES
    mesh = plsc.VectorSubcoreMesh(core_axis_name="c", subcore_axis_name="s")

    @functools.partial(
        pl.kernel,
        out_type=jax.ShapeDtypeStruct((B, _C), jnp.float32),
        mesh=mesh,
        compiler_params=pltpu.CompilerParams(needs_layout_passes=False),
        scratch_types=[
            pltpu.VMEM((L, R), jnp.int32),
            pltpu.VMEM((L, R), jnp.int32),
            pltpu.VMEM((R, _C), jnp.float32),
        ],
    )
    def k(pos_hbm, dep_hbm, out_hbm, pos_v, dep_v, cnt_v):
        wid = lax.axis_index("s") * 2 + lax.axis_index("c")
        base = wid * R
        pltpu.sync_copy(pos_hbm.at[:, pl.ds(base, R)], pos_v)
        pltpu.sync_copy(dep_hbm.at[:, pl.ds(base, R)], dep_v)

        ones = jnp.full((_LANES,), 1.0, jnp.float32)
        zeros = jnp.zeros((_LANES,), jnp.float32)
        iota = lax.iota(jnp.int32, _LANES)

        @plsc.parallel_loop(0, G, unroll=2)
        def grp(g):
            gb = g * _LANES
            rows = gb + iota
            for rr in range(_LANES):
                for j in range(_C // _LANES):
                    cnt_v[gb + rr, pl.ds(j * _LANES, _LANES)] = zeros
            for l in range(L):
                idx = pos_v[l, pl.ds(gb, _LANES)]
                plsc.addupdate_scatter(cnt_v, [rows, idx], ones)
            for l in range(L):
                idx = dep_v[l, pl.ds(gb, _LANES)] + 64
                plsc.addupdate_scatter(cnt_v, [rows, idx], ones)

        pltpu.sync_copy(cnt_v, out_hbm.at[pl.ds(base, R)])

    return k(pos_t, dep_t)


def _tc_matmul(counts, w_pos, w_dep, B):
    BLK = 2048

    def body(c_ref, wp_ref, wd_ref, po_ref, do_ref):
        c = c_ref[...]
        dn = (((0,), (1,)), ((), ()))
        po_ref[...] = lax.dot_general(
            wp_ref[...], c, dn, preferred_element_type=jnp.float32
        )
        do_ref[...] = lax.dot_general(
            wd_ref[...], c, dn, preferred_element_type=jnp.float32
        )

    return pl.pallas_call(
        body,
        grid=(B // BLK,),
        in_specs=[
            pl.BlockSpec((BLK, _C), lambda i: (i, 0)),
            pl.BlockSpec((_C, 64), lambda i: (0, 0)),
            pl.BlockSpec((_C, 64), lambda i: (0, 0)),
        ],
        out_specs=[
            pl.BlockSpec((64, BLK), lambda i: (0, i)),
            pl.BlockSpec((64, BLK), lambda i: (0, i)),
        ],
        out_shape=[
            jax.ShapeDtypeStruct((64, B), jnp.float32),
            jax.ShapeDtypeStruct((64, B), jnp.float32),
        ],
    )(counts, w_pos, w_dep)


def kernel(padded_pos, padded_deprel, pos_table, deprel_table):
    B, L = padded_pos.shape
    counts = _sc_counts(padded_pos.T, padded_deprel.T, B, L)
    w_pos = jnp.zeros((_C, 64), jnp.float32).at[: pos_table.shape[0]].set(pos_table)
    w_dep = (
        jnp.zeros((_C, 64), jnp.float32)
        .at[64 : 64 + deprel_table.shape[0]]
        .set(deprel_table)
    )
    po_t, do_t = _tc_matmul(counts, w_pos, w_dep, B)
    return (po_t.T, do_t.T)


# speedup vs baseline: 146.9348x; 1.0635x over previous
import functools

import jax
import jax.numpy as jnp
from jax import lax
from jax.experimental import pallas as pl
from jax.experimental.pallas import tpu as pltpu
from jax.experimental.pallas import tpu_sc as plsc

_NW = 32
_LANES = 16
_C = 128


def _sc_counts(pos_t, dep_t, B, L):
    R = B // _NW
    G = R // _LANES
    mesh = plsc.VectorSubcoreMesh(core_axis_name="c", subcore_axis_name="s")

    @functools.partial(
        pl.kernel,
        out_type=jax.ShapeDtypeStruct((_C, B), jnp.float32),
        mesh=mesh,
        compiler_params=pltpu.CompilerParams(needs_layout_passes=False),
        scratch_types=[
            pltpu.VMEM((L, R), jnp.int32),
            pltpu.VMEM((L, R), jnp.int32),
            pltpu.VMEM((_C, R), jnp.float32),
        ],
    )
    def k(pos_hbm, dep_hbm, out_hbm, pos_v, dep_v, cnt_v):
        wid = lax.axis_index("s") * 2 + lax.axis_index("c")
        base = wid * R
        pltpu.sync_copy(pos_hbm.at[:, pl.ds(base, R)], pos_v)
        pltpu.sync_copy(dep_hbm.at[:, pl.ds(base, R)], dep_v)

        ones = jnp.full((_LANES,), 1.0, jnp.float32)
        zeros = jnp.zeros((_LANES,), jnp.float32)
        iota = lax.iota(jnp.int32, _LANES)

        @plsc.parallel_loop(0, _C, unroll=2)
        def zrow(c):
            for j in range(G):
                cnt_v[c, pl.ds(j * _LANES, _LANES)] = zeros

        @plsc.parallel_loop(0, G, unroll=2)
        def grp(g):
            gb = g * _LANES
            rows = gb + iota
            for l in range(L):
                idx = pos_v[l, pl.ds(gb, _LANES)]
                plsc.addupdate_scatter(cnt_v, [idx, rows], ones)
            for l in range(L):
                idx = dep_v[l, pl.ds(gb, _LANES)] + 64
                plsc.addupdate_scatter(cnt_v, [idx, rows], ones)

        pltpu.sync_copy(cnt_v, out_hbm.at[:, pl.ds(base, R)])

    return k(pos_t, dep_t)


def _tc_matmul(counts_t, w_pos, w_dep, B):
    BLK = 2048

    def body(c_ref, wp_ref, wd_ref, po_ref, do_ref):
        c = c_ref[...]
        dn = (((0,), (0,)), ((), ()))
        po_ref[...] = lax.dot_general(
            wp_ref[...], c, dn, preferred_element_type=jnp.float32
        )
        do_ref[...] = lax.dot_general(
            wd_ref[...], c, dn, preferred_element_type=jnp.float32
        )

    return pl.pallas_call(
        body,
        grid=(B // BLK,),
        in_specs=[
            pl.BlockSpec((_C, BLK), lambda i: (0, i)),
            pl.BlockSpec((_C, 64), lambda i: (0, 0)),
            pl.BlockSpec((_C, 64), lambda i: (0, 0)),
        ],
        out_specs=[
            pl.BlockSpec((64, BLK), lambda i: (0, i)),
            pl.BlockSpec((64, BLK), lambda i: (0, i)),
        ],
        out_shape=[
            jax.ShapeDtypeStruct((64, B), jnp.float32),
            jax.ShapeDtypeStruct((64, B), jnp.float32),
        ],
    )(counts_t, w_pos, w_dep)


def kernel(padded_pos, padded_deprel, pos_table, deprel_table):
    B, L = padded_pos.shape
    counts_t = _sc_counts(padded_pos.T, padded_deprel.T, B, L)
    w_pos = jnp.zeros((_C, 64), jnp.float32).at[: pos_table.shape[0]].set(pos_table)
    w_dep = (
        jnp.zeros((_C, 64), jnp.float32)
        .at[64 : 64 + deprel_table.shape[0]]
        .set(deprel_table)
    )
    po_t, do_t = _tc_matmul(counts_t, w_pos, w_dep, B)
    return (po_t.T, do_t.T)


# speedup vs baseline: 152.9479x; 1.0409x over previous
import functools

import jax
import jax.numpy as jnp
from jax import lax
from jax.experimental import pallas as pl
from jax.experimental.pallas import tpu as pltpu
from jax.experimental.pallas import tpu_sc as plsc

_NW = 32
_LANES = 16
_CW = 72


def _sc_counts(pos_t, dep_t, B, L):
    R = B // _NW
    CH = R // 2
    GC = CH // _LANES
    mesh = plsc.VectorSubcoreMesh(core_axis_name="c", subcore_axis_name="s")

    @functools.partial(
        pl.kernel,
        out_type=jax.ShapeDtypeStruct((_CW, B), jnp.float32),
        mesh=mesh,
        compiler_params=pltpu.CompilerParams(needs_layout_passes=False),
        scratch_types=[
            pltpu.VMEM((2, L, CH), jnp.int32),
            pltpu.VMEM((2, L, CH), jnp.int32),
            pltpu.VMEM((_CW, R), jnp.float32),
            pltpu.SemaphoreType.DMA,
            pltpu.SemaphoreType.DMA,
        ],
    )
    def k(pos_hbm, dep_hbm, out_hbm, pos_v, dep_v, cnt_v, sem0, sem1):
        wid = lax.axis_index("s") * 2 + lax.axis_index("c")
        base = wid * R
        sems = (sem0, sem1)
        pending = []
        for b in range(2):
            cb = base + b * CH
            pending.append((
                pltpu.async_copy(pos_hbm.at[:, pl.ds(cb, CH)], pos_v.at[b], sems[b]),
                pltpu.async_copy(dep_hbm.at[:, pl.ds(cb, CH)], dep_v.at[b], sems[b]),
            ))

        ones = jnp.full((_LANES,), 1.0, jnp.float32)
        zeros = jnp.zeros((_LANES,), jnp.float32)
        iota = lax.iota(jnp.int32, _LANES)

        @plsc.parallel_loop(0, _CW, unroll=2)
        def zrow(c):
            for j in range(R // _LANES):
                cnt_v[c, pl.ds(j * _LANES, _LANES)] = zeros

        for b in range(2):
            for h in pending[b]:
                h.wait()

            @plsc.parallel_loop(0, GC, unroll=2)
            def grp(g):
                gb = g * _LANES
                rows = b * CH + gb + iota
                for l in range(L):
                    idx = pos_v[b, l, pl.ds(gb, _LANES)]
                    plsc.addupdate_scatter(cnt_v, [idx, rows], ones)
                for l in range(L):
                    idx = dep_v[b, l, pl.ds(gb, _LANES)] + 19
                    plsc.addupdate_scatter(cnt_v, [idx, rows], ones)

        pltpu.sync_copy(cnt_v, out_hbm.at[:, pl.ds(base, R)])

    return k(pos_t, dep_t)


def _tc_matmul(counts_t, w_pos, w_dep, B):
    BLK = 2048

    def body(c_ref, wp_ref, wd_ref, po_ref, do_ref):
        c = c_ref[...]
        dn = (((0,), (0,)), ((), ()))
        po_ref[...] = lax.dot_general(
            wp_ref[...], c, dn, preferred_element_type=jnp.float32
        )
        do_ref[...] = lax.dot_general(
            wd_ref[...], c, dn, preferred_element_type=jnp.float32
        )

    return pl.pallas_call(
        body,
        grid=(B // BLK,),
        in_specs=[
            pl.BlockSpec((_CW, BLK), lambda i: (0, i)),
            pl.BlockSpec((_CW, 64), lambda i: (0, 0)),
            pl.BlockSpec((_CW, 64), lambda i: (0, 0)),
        ],
        out_specs=[
            pl.BlockSpec((64, BLK), lambda i: (0, i)),
            pl.BlockSpec((64, BLK), lambda i: (0, i)),
        ],
        out_shape=[
            jax.ShapeDtypeStruct((64, B), jnp.float32),
            jax.ShapeDtypeStruct((64, B), jnp.float32),
        ],
    )(counts_t, w_pos, w_dep)


def kernel(padded_pos, padded_deprel, pos_table, deprel_table):
    B, L = padded_pos.shape
    counts_t = _sc_counts(padded_pos.T, padded_deprel.T, B, L)
    w_pos = jnp.zeros((_CW, 64), jnp.float32).at[: pos_table.shape[0]].set(pos_table)
    w_dep = (
        jnp.zeros((_CW, 64), jnp.float32)
        .at[19 : 19 + deprel_table.shape[0]]
        .set(deprel_table)
    )
    po_t, do_t = _tc_matmul(counts_t, w_pos, w_dep, B)
    return (po_t.T, do_t.T)
